# Initial kernel scaffold; baseline (speedup 1.0000x reference)
import functools
import numpy as np
import jax
import jax.numpy as jnp
from jax import lax
from jax.experimental import pallas as pl
from jax.experimental.pallas import tpu as pltpu
from jax.experimental.pallas import tpu_sc as plsc

N = 10000
E = 160000
H = 128
L = 4
T = 4
F = 128
TF = T * F
FO = 32
NW = 32
NPW = 320
NPAD = NW * NPW
NCH = 8
CW = TF // NCH
KB = 4000
EB = 512
EPAD = ((E + EB - 1) // EB) * EB
CAP = 16384
AVG_LOG = float(np.log(17.0))
BN_EPS = 1e-5
FINF = jnp.float32(3.0e38)


def _wid():
    return lax.axis_index("s") * 2 + lax.axis_index("c")


def _sc_mesh():
    return plsc.VectorSubcoreMesh(core_axis_name="c", subcore_axis_name="s")


def _csr_body(keys_hbm, vals_hbm, col_hbm, rp_hbm, cnt_hbm,
              keys_v, vals_v, hist_v, rp_v, cur_v, buf_v, cntf_v):
    wid = _wid()
    lo = wid * NPW
    ones = jnp.ones((16,), jnp.int32)

    def inith(i, _):
        hist_v[pl.ds(i * 16, 16)] = jnp.zeros((16,), jnp.int32)
        return 0
    lax.fori_loop(0, NPW // 16, inith, 0)

    def p1_block(b, _):
        pltpu.sync_copy(keys_hbm.at[pl.ds(b * KB, KB)], keys_v)

        def p1_vec(i, _):
            k = keys_v[pl.ds(i * 16, 16)]
            m = (k >= lo) & (k < lo + NPW)
            kl = jnp.clip(k - lo, 0, NPW - 1)
            plsc.addupdate_scatter(hist_v, [kl], ones, mask=m)
            return 0
        lax.fori_loop(0, KB // 16, p1_vec, 0)
        return 0
    lax.fori_loop(0, E // KB, p1_block, 0)

    def psum(j, carry):
        v = hist_v[pl.ds(j * 16, 16)]
        c = plsc.cumsum(v)
        rp_v[pl.ds(j * 16, 16)] = carry + c - v
        return carry + lax.reduce_max(c, (0,))
    total = lax.fori_loop(0, NPW // 16, psum, jnp.int32(0))
    lane = lax.iota(jnp.int32, 16)
    rp_v[pl.ds(NPW, 16)] = jnp.where(lane == 0, total, 0)

    pltpu.sync_copy(rp_v, rp_hbm.at[wid])

    def cdeg(j, _):
        cntf_v[pl.ds(j * 16, 16)] = hist_v[pl.ds(j * 16, 16)].astype(jnp.float32)
        return 0
    lax.fori_loop(0, NPW // 16, cdeg, 0)
    pltpu.sync_copy(cntf_v, cnt_hbm.at[pl.ds(lo, NPW)])

    cal, _ = plsc.scan_count(jnp.zeros((16,), jnp.int32))
    r0 = lax.reduce_min(cal, (0,))

    nwin = (total + CAP - 1) // CAP

    def window(w, _):
        wbase = w * CAP

        def zero(i, _):
            buf_v[pl.ds(i * 16, 16)] = jnp.zeros((16,), jnp.int32)
            return 0
        lax.fori_loop(0, CAP // 16, zero, 0)

        def rcur(j, _):
            cur_v[pl.ds(j * 16, 16)] = rp_v[pl.ds(j * 16, 16)]
            return 0
        lax.fori_loop(0, 336 // 16, rcur, 0)

        def p2_block(b, _):
            pltpu.sync_copy(keys_hbm.at[pl.ds(b * KB, KB)], keys_v)
            pltpu.sync_copy(vals_hbm.at[pl.ds(b * KB, KB)], vals_v)

            def p2_vec(i, _):
                k = keys_v[pl.ds(i * 16, 16)]
                v = vals_v[pl.ds(i * 16, 16)]
                m = (k >= lo) & (k < lo + NPW)
                kl = jnp.where(m, jnp.clip(k - lo, 0, NPW - 1), NPW)
                rank, lastm = plsc.scan_count(kl, mask=m)
                base = plsc.load_gather(cur_v, [kl], mask=m)
                pos = base + rank - r0
                mw = m & (pos >= wbase) & (pos < wbase + CAP)
                plsc.store_scatter(buf_v, [jnp.clip(pos - wbase, 0, CAP - 1)],
                                   v, mask=mw)
                plsc.store_scatter(cur_v, [kl], pos + 1, mask=lastm & m)
                return 0
            lax.fori_loop(0, KB // 16, p2_vec, 0)
            return 0
        lax.fori_loop(0, E // KB, p2_block, 0)

        wlen = jnp.minimum(CAP, ((total - wbase + 7) // 8) * 8)
        pltpu.sync_copy(buf_v.at[pl.ds(0, wlen)],
                        col_hbm.at[wid].at[pl.ds(wbase, wlen)])
        return 0
    lax.fori_loop(0, nwin, window, 0)

    def zero16(i, _):
        buf_v[pl.ds(i * 16, 16)] = jnp.zeros((16,), jnp.int32)
        return 0
    lax.fori_loop(0, EB // 16, zero16, 0)
    start = ((total + 7) // 8) * 8
    end = ((total + EB - 1) // EB) * EB
    plen = end - start

    @pl.when(plen > 0)
    def _():
        pltpu.sync_copy(buf_v.at[pl.ds(0, plen)],
                        col_hbm.at[wid].at[pl.ds(start, plen)])


def _build_csr(keys, vals):
    fn = pl.kernel(
        _csr_body,
        out_type=[
            jax.ShapeDtypeStruct((NW, EPAD), jnp.int32),
            jax.ShapeDtypeStruct((NW, 336), jnp.int32),
            jax.ShapeDtypeStruct((NPAD,), jnp.float32),
        ],
        mesh=_sc_mesh(),
        scratch_types=[
            pltpu.VMEM((KB,), jnp.int32),
            pltpu.VMEM((KB,), jnp.int32),
            pltpu.VMEM((NPW,), jnp.int32),
            pltpu.VMEM((336,), jnp.int32),
            pltpu.VMEM((336,), jnp.int32),
            pltpu.VMEM((CAP,), jnp.int32),
            pltpu.VMEM((NPW,), jnp.float32),
        ],
    )
    return fn(keys, vals)


def _seg_body(b3_hbm, col_hbm, rp_hbm, s1_hbm, s2_hbm, mn_hbm, mx_hbm,
              rp_v, rp_s, colv, rows, accS, accQ, accMn, accMx, sem):
    wid = _wid()
    pltpu.sync_copy(rp_hbm.at[wid], rp_v)
    pltpu.sync_copy(rp_v, rp_s)
    cw = rp_s[NPW]
    nblk = (cw + EB - 1) // EB

    def chunk(c, _):
        def initacc(i, _):
            z = jnp.zeros((16,), jnp.float32)
            r = i // 4
            k = (i % 4) * 16
            accS[r, pl.ds(k, 16)] = z
            accQ[r, pl.ds(k, 16)] = z
            accMn[r, pl.ds(k, 16)] = jnp.full((16,), FINF, jnp.float32)
            accMx[r, pl.ds(k, 16)] = jnp.full((16,), -FINF, jnp.float32)
            return 0
        lax.fori_loop(0, NPW * 4, initacc, 0)

        def block(eb, n0):
            e0 = eb * EB
            e1 = jnp.minimum(e0 + EB, cw)
            pltpu.sync_copy(col_hbm.at[wid].at[pl.ds(e0, EB)], colv)
            for g in range(EB // 128):
                pltpu.async_copy(
                    b3_hbm.at[c].at[colv.at[pl.ds(g * 128, 128)]],
                    rows.at[pl.ds(g * 128, 128)], sem).wait()

            def node_cond(carry):
                n, done = carry
                return jnp.logical_not(done) & (n < NPW)

            def node_body(carry):
                n, _ = carry
                rs = jnp.maximum(rp_s[n], e0)
                re = jnp.minimum(rp_s[n + 1], e1)
                a = [accS[n, pl.ds(k * 16, 16)] for k in range(4)]
                q = [accQ[n, pl.ds(k * 16, 16)] for k in range(4)]
                mn = [accMn[n, pl.ds(k * 16, 16)] for k in range(4)]
                mx = [accMx[n, pl.ds(k * 16, 16)] for k in range(4)]

                def edge(e, st):
                    sa, sq, smn, smx = st
                    r = e - e0
                    v = [rows[r, pl.ds(k * 16, 16)] for k in range(4)]
                    sa = [sa[k] + v[k] for k in range(4)]
                    sq = [sq[k] + v[k] * v[k] for k in range(4)]
                    smn = [jnp.minimum(smn[k], v[k]) for k in range(4)]
                    smx = [jnp.maximum(smx[k], v[k]) for k in range(4)]
                    return (sa, sq, smn, smx)
                a, q, mn, mx = lax.fori_loop(rs, jnp.maximum(rs, re), edge,
                                             (a, q, mn, mx))
                for k in range(4):
                    accS[n, pl.ds(k * 16, 16)] = a[k]
                    accQ[n, pl.ds(k * 16, 16)] = q[k]
                    accMn[n, pl.ds(k * 16, 16)] = mn[k]
                    accMx[n, pl.ds(k * 16, 16)] = mx[k]
                adv = rp_s[n + 1] <= e1
                return (jnp.where(adv, n + 1, n), jnp.logical_not(adv))

            nfin, _ = lax.while_loop(node_cond, node_body, (n0, cw <= e0))
            return nfin
        lax.fori_loop(0, nblk, block, jnp.int32(0))

        lo = wid * NPW
        pltpu.sync_copy(accS, s1_hbm.at[pl.ds(lo, NPW), pl.ds(c * CW, CW)])
        pltpu.sync_copy(accQ, s2_hbm.at[pl.ds(lo, NPW), pl.ds(c * CW, CW)])
        pltpu.sync_copy(accMn, mn_hbm.at[pl.ds(lo, NPW), pl.ds(c * CW, CW)])
        pltpu.sync_copy(accMx, mx_hbm.at[pl.ds(lo, NPW), pl.ds(c * CW, CW)])
        return 0
    lax.fori_loop(0, NCH, chunk, 0)


def _segment_reduce(b3, col, rp):
    fn = pl.kernel(
        _seg_body,
        out_type=[jax.ShapeDtypeStruct((NPAD, TF), jnp.float32)
                  for _ in range(4)],
        mesh=_sc_mesh(),
        scratch_types=[
            pltpu.VMEM((336,), jnp.int32),
            pltpu.SMEM((336,), jnp.int32),
            pltpu.VMEM((EB,), jnp.int32),
            pltpu.VMEM((EB, CW), jnp.float32),
            pltpu.VMEM((NPW, CW), jnp.float32),
            pltpu.VMEM((NPW, CW), jnp.float32),
            pltpu.VMEM((NPW, CW), jnp.float32),
            pltpu.VMEM((NPW, CW), jnp.float32),
            pltpu.SemaphoreType.DMA,
        ],
    )
    return fn(b3, col, rp)


RB = 256


def _in_body(x_ref, w_ref, b_ref, o_ref):
    o_ref[...] = jax.nn.relu(
        jnp.dot(x_ref[...], w_ref[...], preferred_element_type=jnp.float32)
        + b_ref[...])


def _input_proj(x, w_t, b):
    return pl.pallas_call(
        _in_body,
        grid=(NPAD // RB,),
        in_specs=[
            pl.BlockSpec((RB, H), lambda i: (i, 0)),
            pl.BlockSpec((H, H), lambda i: (0, 0)),
            pl.BlockSpec((1, H), lambda i: (0, 0)),
        ],
        out_specs=pl.BlockSpec((RB, H), lambda i: (i, 0)),
        out_shape=jax.ShapeDtypeStruct((NPAD, H), jnp.float32),
    )(x, w_t, b)


def _pre_body(h_ref, wa_ref, wb_ref, pb_ref, a_ref, b3_ref):
    h = h_ref[...]
    a_ref[...] = jnp.dot(h, wa_ref[...],
                         preferred_element_type=jnp.float32) + pb_ref[...]
    b3_ref[0] = jnp.dot(h, wb_ref[...], preferred_element_type=jnp.float32)


def _pre_proj(h, wa, wb, pb):
    return pl.pallas_call(
        _pre_body,
        grid=(NPAD // RB, NCH),
        in_specs=[
            pl.BlockSpec((RB, H), lambda i, j: (i, 0)),
            pl.BlockSpec((H, CW), lambda i, j: (0, j)),
            pl.BlockSpec((H, CW), lambda i, j: (0, j)),
            pl.BlockSpec((1, CW), lambda i, j: (0, j)),
        ],
        out_specs=[
            pl.BlockSpec((RB, CW), lambda i, j: (i, j)),
            pl.BlockSpec((1, RB, CW), lambda i, j: (j, i, 0)),
        ],
        out_shape=[
            jax.ShapeDtypeStruct((NPAD, TF), jnp.float32),
            jax.ShapeDtypeStruct((NCH, NPAD, CW), jnp.float32),
        ],
    )(h, wa, wb, pb)


PB = 320


def _post_body(a_ref, s1_ref, s2_ref, mn_ref, mx_ref, cnt_ref, h_ref,
               wsc_ref, wfix_ref, lin_ref, bias_ref, o_ref):
    A = a_ref[...]
    S1 = s1_ref[...]
    cnt = cnt_ref[...]
    deg = jnp.maximum(cnt, 1.0)
    mean = (cnt * A + S1) / deg
    msq = (cnt * A * A + 2.0 * A * S1 + s2_ref[...]) / deg
    std = jnp.sqrt(jnp.maximum(msq - mean * mean, 0.0) + 1e-5)
    he = cnt > 0.0
    mn = jnp.where(he, A + mn_ref[...], 0.0)
    mx = jnp.where(he, A + mx_ref[...], 0.0)
    logd = jnp.log(deg + 1.0)
    c1 = logd * (1.0 / AVG_LOG)
    c2 = AVG_LOG / logd
    h = h_ref[...]
    outs = []
    for t in range(T):
        sl = slice(t * F, (t + 1) * F)
        agg = jnp.concatenate([mean[:, sl], mn[:, sl], mx[:, sl], std[:, sl]],
                              axis=1)
        psc = jnp.dot(agg, wsc_ref[t], preferred_element_type=jnp.float32)
        pfix = jnp.dot(jnp.concatenate([h, agg], axis=1), wfix_ref[t],
                       preferred_element_type=jnp.float32)
        outs.append(pfix + c1 * psc[:, :FO] + c2 * psc[:, FO:])
    out = jnp.concatenate(outs, axis=1)
    o_ref[...] = jax.nn.relu(
        jnp.dot(out, lin_ref[...], preferred_element_type=jnp.float32)
        + bias_ref[...])


def _post(a, s1, s2, mn, mx, cnt2, h, wsc, wfix, lin2, bias2):
    return pl.pallas_call(
        _post_body,
        grid=(NPAD // PB,),
        in_specs=[
            pl.BlockSpec((PB, TF), lambda i: (i, 0)),
            pl.BlockSpec((PB, TF), lambda i: (i, 0)),
            pl.BlockSpec((PB, TF), lambda i: (i, 0)),
            pl.BlockSpec((PB, TF), lambda i: (i, 0)),
            pl.BlockSpec((PB, TF), lambda i: (i, 0)),
            pl.BlockSpec((PB, 1), lambda i: (i, 0)),
            pl.BlockSpec((PB, H), lambda i: (i, 0)),
            pl.BlockSpec((T, 4 * F, 2 * FO), lambda i: (0, 0, 0)),
            pl.BlockSpec((T, H + 4 * F, FO), lambda i: (0, 0, 0)),
            pl.BlockSpec((H, H), lambda i: (0, 0)),
            pl.BlockSpec((1, H), lambda i: (0, 0)),
        ],
        out_specs=pl.BlockSpec((PB, H), lambda i: (i, 0)),
        out_shape=jax.ShapeDtypeStruct((NPAD, H), jnp.float32),
    )(a, s1, s2, mn, mx, cnt2, h, wsc, wfix, lin2, bias2)


def _mlp_body(h_ref, w1_ref, b1_ref, w2_ref, b2_ref, o_ref):
    t = jax.nn.relu(
        jnp.dot(h_ref[...], w1_ref[...], preferred_element_type=jnp.float32)
        + b1_ref[...])
    o_ref[...] = jnp.dot(t, w2_ref[...],
                         preferred_element_type=jnp.float32) + b2_ref[...]


def _mlp(h, w1t, b1, w2t, b2):
    return pl.pallas_call(
        _mlp_body,
        grid=(NPAD // RB,),
        in_specs=[
            pl.BlockSpec((RB, H), lambda i: (i, 0)),
            pl.BlockSpec((H, H), lambda i: (0, 0)),
            pl.BlockSpec((1, H), lambda i: (0, 0)),
            pl.BlockSpec((H, H), lambda i: (0, 0)),
            pl.BlockSpec((1, H), lambda i: (0, 0)),
        ],
        out_specs=pl.BlockSpec((RB, H), lambda i: (i, 0)),
        out_shape=jax.ShapeDtypeStruct((NPAD, H), jnp.float32),
    )(h, w1t, b1, w2t, b2)


def kernel(x, edge_index, W_in, b_in, pre_W, pre_b, post_W, post_b,
           lin_W, lin_b, bn_w, bn_b, mlp_W1, mlp_b1, mlp_W2, mlp_b2):
    x = x.astype(jnp.float32)
    xp = jnp.pad(x, ((0, NPAD - N), (0, 0)))
    src = edge_index[0].astype(jnp.int32)
    dst = edge_index[1].astype(jnp.int32)

    col_f, rp_f, cnt_f = _build_csr(dst, src)
    col_b, rp_b, cnt_b = _build_csr(src, dst)

    h = _input_proj(xp, W_in.T, b_in.reshape(1, H))

    dirs = ['f', 'f', 'b', 'b']
    bn_scale = (bn_w / np.sqrt(1.0 + BN_EPS)).astype(jnp.float32)
    for l in range(L):
        col, rp, cnt = (col_f, rp_f, cnt_f) if dirs[l] == 'f' else \
                       (col_b, rp_b, cnt_b)
        preW = pre_W[l]
        wa = preW[:, :, :F].transpose(2, 0, 1).reshape(H, TF)
        wb = preW[:, :, F:].transpose(2, 0, 1).reshape(H, TF)
        pb = pre_b[l].reshape(1, TF)
        a, b3 = _pre_proj(h, wa, wb, pb)
        s1, s2, smn, smx = _segment_reduce(b3, col, rp)

        pw = post_W[l]
        wx = pw[:, :, :F]
        wamp = pw[:, :, F:5 * F]
        watt = pw[:, :, 5 * F:9 * F]
        wid_ = pw[:, :, 9 * F:]
        wsc = jnp.concatenate([wamp.transpose(0, 2, 1),
                               watt.transpose(0, 2, 1)], axis=2)
        wfix = jnp.concatenate([wx, wid_], axis=2).transpose(0, 2, 1)
        bias = (post_b[l].reshape(-1) @ lin_W[l].T + lin_b[l])
        lin2 = lin_W[l].T * bn_scale[l][None, :]
        bias2 = (bias * bn_scale[l] + bn_b[l]).reshape(1, H)
        h = _post(a, s1, s2, smn, smx, cnt.reshape(NPAD, 1), h,
                  wsc, wfix, lin2, bias2)

    out = _mlp(h, mlp_W1.T, mlp_b1.reshape(1, H), mlp_W2.T,
               mlp_b2.reshape(1, H))
    return out[:N]


# speedup vs baseline: 49.6577x; 49.6577x over previous
import functools
import numpy as np
import jax
import jax.numpy as jnp
from jax import lax
from jax.experimental import pallas as pl
from jax.experimental.pallas import tpu as pltpu
from jax.experimental.pallas import tpu_sc as plsc

N = 10000
E = 160000
H = 128
L = 4
T = 4
F = 128
TF = T * F
FO = 32
NW = 32
NPW = 320
NPAD = NW * NPW
NCH = 4
CW = TF // NCH
NHALF = 2
NPH = NPW // NHALF
KB = 4000
EB = 256
CAP = 16384
EPAD = ((E + CAP - 1) // CAP) * CAP
AVG_LOG = float(np.log(17.0))
BN_EPS = 1e-5
FINF = 3.0e38


def _wid():
    return lax.axis_index("s") * 2 + lax.axis_index("c")


def _sc_mesh():
    return plsc.VectorSubcoreMesh(core_axis_name="c", subcore_axis_name="s")


def _csr_body(keys_hbm, vals_hbm, col_hbm, rp_hbm, cnt_hbm,
              keys_v, vals_v, hist_v, rp_v, cur_v, buf_v, cntf_v):
    wid = _wid()
    lo = wid * NPW
    ones = jnp.ones((16,), jnp.int32)

    def inith(i, _):
        hist_v[pl.ds(i * 16, 16)] = jnp.zeros((16,), jnp.int32)
        return 0
    lax.fori_loop(0, NPW // 16, inith, 0)

    def p1_block(b, _):
        pltpu.sync_copy(keys_hbm.at[pl.ds(b * KB, KB)], keys_v)

        def p1_vec(i, _):
            k = keys_v[pl.ds(i * 16, 16)]
            m = (k >= lo) & (k < lo + NPW)
            kl = jnp.clip(k - lo, 0, NPW - 1)
            plsc.addupdate_scatter(hist_v, [kl], ones, mask=m)
            return 0
        lax.fori_loop(0, KB // 16, p1_vec, 0)
        return 0
    lax.fori_loop(0, E // KB, p1_block, 0)

    def psum(j, carry):
        v = hist_v[pl.ds(j * 16, 16)]
        c = plsc.cumsum(v)
        rp_v[pl.ds(j * 16, 16)] = carry + c - v
        return carry + lax.reduce_max(c, (0,))
    total = lax.fori_loop(0, NPW // 16, psum, jnp.int32(0))
    lane = lax.iota(jnp.int32, 16)
    rp_v[pl.ds(NPW, 16)] = jnp.where(lane == 0, total, 0)

    pltpu.sync_copy(rp_v, rp_hbm.at[wid])

    def cdeg(j, _):
        cntf_v[pl.ds(j * 16, 16)] = hist_v[pl.ds(j * 16, 16)].astype(jnp.float32)
        return 0
    lax.fori_loop(0, NPW // 16, cdeg, 0)
    pltpu.sync_copy(cntf_v, cnt_hbm.at[pl.ds(lo, NPW)])

    cal, _ = plsc.scan_count(jnp.zeros((16,), jnp.int32))
    r0 = lax.reduce_min(cal, (0,))

    nwin = (total + CAP - 1) // CAP

    def window(w, _):
        wbase = w * CAP

        def zero(i, _):
            buf_v[pl.ds(i * 16, 16)] = jnp.zeros((16,), jnp.int32)
            return 0
        lax.fori_loop(0, CAP // 16, zero, 0)

        def rcur(j, _):
            cur_v[pl.ds(j * 16, 16)] = rp_v[pl.ds(j * 16, 16)]
            return 0
        lax.fori_loop(0, 336 // 16, rcur, 0)

        def p2_block(b, _):
            pltpu.sync_copy(keys_hbm.at[pl.ds(b * KB, KB)], keys_v)
            pltpu.sync_copy(vals_hbm.at[pl.ds(b * KB, KB)], vals_v)

            def p2_vec(i, _):
                k = keys_v[pl.ds(i * 16, 16)]
                v = vals_v[pl.ds(i * 16, 16)]
                m = (k >= lo) & (k < lo + NPW)
                kl = jnp.where(m, jnp.clip(k - lo, 0, NPW - 1), NPW)
                rank, lastm = plsc.scan_count(kl, mask=m)
                base = plsc.load_gather(cur_v, [kl], mask=m)
                pos = base + rank - r0
                mw = m & (pos >= wbase) & (pos < wbase + CAP)
                plsc.store_scatter(buf_v, [jnp.clip(pos - wbase, 0, CAP - 1)],
                                   v, mask=mw)
                plsc.store_scatter(cur_v, [kl], pos + 1, mask=lastm & m)
                return 0
            lax.fori_loop(0, KB // 16, p2_vec, 0)
            return 0
        lax.fori_loop(0, E // KB, p2_block, 0)

        pltpu.sync_copy(buf_v, col_hbm.at[wid].at[pl.ds(wbase, CAP)])
        return 0
    lax.fori_loop(0, nwin, window, 0)


def _build_csr(keys, vals):
    fn = pl.kernel(
        _csr_body,
        out_type=[
            jax.ShapeDtypeStruct((NW, EPAD), jnp.int32),
            jax.ShapeDtypeStruct((NW, 336), jnp.int32),
            jax.ShapeDtypeStruct((NPAD,), jnp.float32),
        ],
        mesh=_sc_mesh(),
        compiler_params=pltpu.CompilerParams(needs_layout_passes=False),
        scratch_types=[
            pltpu.VMEM((KB,), jnp.int32),
            pltpu.VMEM((KB,), jnp.int32),
            pltpu.VMEM((NPW,), jnp.int32),
            pltpu.VMEM((336,), jnp.int32),
            pltpu.VMEM((336,), jnp.int32),
            pltpu.VMEM((CAP,), jnp.int32),
            pltpu.VMEM((NPW,), jnp.float32),
        ],
    )
    return fn(keys, vals)


def _seg_body(b3_hbm, col_hbm, rp_hbm, s1_hbm, s2_hbm, mn_hbm, mx_hbm,
              rp_s, colv, rows, accS, accQ, accMn, accMx, sem):
    wid = _wid()
    pltpu.sync_copy(rp_hbm.at[wid], rp_s)

    def rd(i):
        return rp_s[pl.ds(i, 16)][0]
    NV = CW // 16

    def chunk(ch, _):
        c = ch // NHALF
        half = ch % NHALF
        n_lo = half * NPH
        n_hi = n_lo + NPH
        lo_e = rd(n_lo)
        hi_e = rd(n_hi)

        def initacc(i, _):
            z = jnp.zeros((16,), jnp.float32)
            r = i // NV
            k = (i % NV) * 16
            accS[r, pl.ds(k, 16)] = z
            accQ[r, pl.ds(k, 16)] = z
            accMn[r, pl.ds(k, 16)] = jnp.full((16,), FINF, jnp.float32)
            accMx[r, pl.ds(k, 16)] = jnp.full((16,), -FINF, jnp.float32)
            return 0
        lax.fori_loop(0, NPH * NV, initacc, 0)

        b0 = lo_e // EB
        b1 = (hi_e + EB - 1) // EB

        def block(eb, n0):
            e0 = eb * EB
            e1 = jnp.minimum(e0 + EB, hi_e)
            pltpu.sync_copy(col_hbm.at[wid].at[pl.ds(e0, EB)], colv)
            for g in range(EB // 128):
                pltpu.async_copy(
                    b3_hbm.at[c].at[colv.at[pl.ds(g * 128, 128)]],
                    rows.at[pl.ds(g * 128, 128)], sem).wait()

            def node_cond(carry):
                n, done = carry
                return jnp.logical_not(done) & (n < n_hi)

            def node_body(carry):
                n, _ = carry
                na = n - n_lo
                rs = jnp.maximum(rd(n), e0)
                re = jnp.minimum(rd(n + 1), e1)
                a = [accS[na, pl.ds(k * 16, 16)] for k in range(NV)]
                q = [accQ[na, pl.ds(k * 16, 16)] for k in range(NV)]
                mn = [accMn[na, pl.ds(k * 16, 16)] for k in range(NV)]
                mx = [accMx[na, pl.ds(k * 16, 16)] for k in range(NV)]

                def edge(e, st):
                    sa, sq, smn, smx = st
                    r = e - e0
                    v = [rows[r, pl.ds(k * 16, 16)] for k in range(NV)]
                    sa = [sa[k] + v[k] for k in range(NV)]
                    sq = [sq[k] + v[k] * v[k] for k in range(NV)]
                    smn = [jnp.minimum(smn[k], v[k]) for k in range(NV)]
                    smx = [jnp.maximum(smx[k], v[k]) for k in range(NV)]
                    return (sa, sq, smn, smx)
                a, q, mn, mx = lax.fori_loop(rs, jnp.maximum(rs, re), edge,
                                             (a, q, mn, mx))
                for k in range(NV):
                    accS[na, pl.ds(k * 16, 16)] = a[k]
                    accQ[na, pl.ds(k * 16, 16)] = q[k]
                    accMn[na, pl.ds(k * 16, 16)] = mn[k]
                    accMx[na, pl.ds(k * 16, 16)] = mx[k]
                adv = rd(n + 1) <= e1
                return (jnp.where(adv, n + 1, n), jnp.logical_not(adv))

            nfin, _ = lax.while_loop(node_cond, node_body,
                                     (n0, hi_e <= e0))
            return nfin
        lax.fori_loop(b0, b1, block, n_lo)

        lo = wid * NPW + n_lo
        pltpu.sync_copy(accS, s1_hbm.at[c].at[pl.ds(lo, NPH)])
        pltpu.sync_copy(accQ, s2_hbm.at[c].at[pl.ds(lo, NPH)])
        pltpu.sync_copy(accMn, mn_hbm.at[c].at[pl.ds(lo, NPH)])
        pltpu.sync_copy(accMx, mx_hbm.at[c].at[pl.ds(lo, NPH)])
        return 0
    lax.fori_loop(0, NCH * NHALF, chunk, 0)


def _segment_reduce(b3, col, rp):
    fn = pl.kernel(
        _seg_body,
        out_type=[jax.ShapeDtypeStruct((NCH, NPAD, CW), jnp.float32)
                  for _ in range(4)],
        mesh=_sc_mesh(),
        compiler_params=pltpu.CompilerParams(needs_layout_passes=False),
        scratch_types=[
            pltpu.VMEM((336,), jnp.int32),
            pltpu.VMEM((EB,), jnp.int32),
            pltpu.VMEM((EB, CW), jnp.float32),
            pltpu.VMEM((NPH, CW), jnp.float32),
            pltpu.VMEM((NPH, CW), jnp.float32),
            pltpu.VMEM((NPH, CW), jnp.float32),
            pltpu.VMEM((NPH, CW), jnp.float32),
            pltpu.SemaphoreType.DMA,
        ],
    )
    return fn(b3, col, rp)


RB = 256


def _in_body(x_ref, w_ref, b_ref, o_ref):
    o_ref[...] = jax.nn.relu(
        jnp.dot(x_ref[...], w_ref[...], preferred_element_type=jnp.float32)
        + b_ref[...])


def _input_proj(x, w_t, b):
    return pl.pallas_call(
        _in_body,
        grid=(NPAD // RB,),
        in_specs=[
            pl.BlockSpec((RB, H), lambda i: (i, 0)),
            pl.BlockSpec((H, H), lambda i: (0, 0)),
            pl.BlockSpec((1, H), lambda i: (0, 0)),
        ],
        out_specs=pl.BlockSpec((RB, H), lambda i: (i, 0)),
        out_shape=jax.ShapeDtypeStruct((NPAD, H), jnp.float32),
    )(x, w_t, b)


def _pre_body(h_ref, wa_ref, wb_ref, pb_ref, a_ref, b3_ref):
    h = h_ref[...]
    a_ref[...] = jnp.dot(h, wa_ref[...],
                         preferred_element_type=jnp.float32) + pb_ref[...]
    b3_ref[0] = jnp.dot(h, wb_ref[...], preferred_element_type=jnp.float32)


def _pre_proj(h, wa, wb, pb):
    return pl.pallas_call(
        _pre_body,
        grid=(NPAD // RB, NCH),
        in_specs=[
            pl.BlockSpec((RB, H), lambda i, j: (i, 0)),
            pl.BlockSpec((H, CW), lambda i, j: (0, j)),
            pl.BlockSpec((H, CW), lambda i, j: (0, j)),
            pl.BlockSpec((1, CW), lambda i, j: (0, j)),
        ],
        out_specs=[
            pl.BlockSpec((RB, CW), lambda i, j: (i, j)),
            pl.BlockSpec((1, RB, CW), lambda i, j: (j, i, 0)),
        ],
        out_shape=[
            jax.ShapeDtypeStruct((NPAD, TF), jnp.float32),
            jax.ShapeDtypeStruct((NCH, NPAD, CW), jnp.float32),
        ],
    )(h, wa, wb, pb)


PB = 320


def _post_body(a_ref, s1_ref, s2_ref, mn_ref, mx_ref, cnt_ref, h_ref,
               wsc_ref, wfix_ref, lin_ref, bias_ref, o_ref):
    cnt = cnt_ref[...]
    deg = jnp.maximum(cnt, 1.0)
    he = cnt > 0.0
    logd = jnp.log(deg + 1.0)
    c1 = logd * (1.0 / AVG_LOG)
    c2 = AVG_LOG / logd
    h = h_ref[...]
    outs = []
    CPT = F // CW
    for t in range(T):
        mean_c, mn_c, mx_c, std_c = [], [], [], []
        for cc in range(CPT):
            c = t * CPT + cc
            A = a_ref[:, pl.ds((t * CPT + cc) * CW, CW)]
            S1 = s1_ref[c]
            mean = (cnt * A + S1) / deg
            msq = (cnt * A * A + 2.0 * A * S1 + s2_ref[c]) / deg
            std = jnp.sqrt(jnp.maximum(msq - mean * mean, 0.0) + 1e-5)
            mean_c.append(mean)
            std_c.append(std)
            mn_c.append(jnp.where(he, A + mn_ref[c], 0.0))
            mx_c.append(jnp.where(he, A + mx_ref[c], 0.0))
        agg = jnp.concatenate(mean_c + mn_c + mx_c + std_c, axis=1)
        psc = jnp.dot(agg, wsc_ref[t], preferred_element_type=jnp.float32)
        pfix = jnp.dot(jnp.concatenate([h, agg], axis=1), wfix_ref[t],
                       preferred_element_type=jnp.float32)
        outs.append(pfix + c1 * psc[:, :FO] + c2 * psc[:, FO:])
    out = jnp.concatenate(outs, axis=1)
    o_ref[...] = jax.nn.relu(
        jnp.dot(out, lin_ref[...], preferred_element_type=jnp.float32)
        + bias_ref[...])


def _post(a, s1, s2, mn, mx, cnt2, h, wsc, wfix, lin2, bias2):
    return pl.pallas_call(
        _post_body,
        grid=(NPAD // PB,),
        in_specs=[
            pl.BlockSpec((PB, TF), lambda i: (i, 0)),
            pl.BlockSpec((NCH, PB, CW), lambda i: (0, i, 0)),
            pl.BlockSpec((NCH, PB, CW), lambda i: (0, i, 0)),
            pl.BlockSpec((NCH, PB, CW), lambda i: (0, i, 0)),
            pl.BlockSpec((NCH, PB, CW), lambda i: (0, i, 0)),
            pl.BlockSpec((PB, 1), lambda i: (i, 0)),
            pl.BlockSpec((PB, H), lambda i: (i, 0)),
            pl.BlockSpec((T, 4 * F, 2 * FO), lambda i: (0, 0, 0)),
            pl.BlockSpec((T, H + 4 * F, FO), lambda i: (0, 0, 0)),
            pl.BlockSpec((H, H), lambda i: (0, 0)),
            pl.BlockSpec((1, H), lambda i: (0, 0)),
        ],
        out_specs=pl.BlockSpec((PB, H), lambda i: (i, 0)),
        out_shape=jax.ShapeDtypeStruct((NPAD, H), jnp.float32),
    )(a, s1, s2, mn, mx, cnt2, h, wsc, wfix, lin2, bias2)


def _mlp_body(h_ref, w1_ref, b1_ref, w2_ref, b2_ref, o_ref):
    t = jax.nn.relu(
        jnp.dot(h_ref[...], w1_ref[...], preferred_element_type=jnp.float32)
        + b1_ref[...])
    o_ref[...] = jnp.dot(t, w2_ref[...],
                         preferred_element_type=jnp.float32) + b2_ref[...]


def _mlp(h, w1t, b1, w2t, b2):
    return pl.pallas_call(
        _mlp_body,
        grid=(NPAD // RB,),
        in_specs=[
            pl.BlockSpec((RB, H), lambda i: (i, 0)),
            pl.BlockSpec((H, H), lambda i: (0, 0)),
            pl.BlockSpec((1, H), lambda i: (0, 0)),
            pl.BlockSpec((H, H), lambda i: (0, 0)),
            pl.BlockSpec((1, H), lambda i: (0, 0)),
        ],
        out_specs=pl.BlockSpec((RB, H), lambda i: (i, 0)),
        out_shape=jax.ShapeDtypeStruct((NPAD, H), jnp.float32),
    )(h, w1t, b1, w2t, b2)


def kernel(x, edge_index, W_in, b_in, pre_W, pre_b, post_W, post_b,
           lin_W, lin_b, bn_w, bn_b, mlp_W1, mlp_b1, mlp_W2, mlp_b2):
    x = x.astype(jnp.float32)
    xp = jnp.pad(x, ((0, NPAD - N), (0, 0)))
    src = edge_index[0].astype(jnp.int32)
    dst = edge_index[1].astype(jnp.int32)

    col_f, rp_f, cnt_f = _build_csr(dst, src)
    col_b, rp_b, cnt_b = _build_csr(src, dst)

    h = _input_proj(xp, W_in.T, b_in.reshape(1, H))

    dirs = ['f', 'f', 'b', 'b']
    bn_scale = (bn_w / np.sqrt(1.0 + BN_EPS)).astype(jnp.float32)
    for l in range(L):
        col, rp, cnt = (col_f, rp_f, cnt_f) if dirs[l] == 'f' else \
                       (col_b, rp_b, cnt_b)
        preW = pre_W[l]
        wa = preW[:, :, :F].transpose(2, 0, 1).reshape(H, TF)
        wb = preW[:, :, F:].transpose(2, 0, 1).reshape(H, TF)
        pb = pre_b[l].reshape(1, TF)
        a, b3 = _pre_proj(h, wa, wb, pb)
        s1, s2, smn, smx = _segment_reduce(b3, col, rp)

        pw = post_W[l]
        wx = pw[:, :, :F]
        wamp = pw[:, :, F:5 * F]
        watt = pw[:, :, 5 * F:9 * F]
        wid_ = pw[:, :, 9 * F:]
        wsc = jnp.concatenate([wamp.transpose(0, 2, 1),
                               watt.transpose(0, 2, 1)], axis=2)
        wfix = jnp.concatenate([wx, wid_], axis=2).transpose(0, 2, 1)
        bias = (post_b[l].reshape(-1) @ lin_W[l].T + lin_b[l])
        lin2 = lin_W[l].T * bn_scale[l][None, :]
        bias2 = (bias * bn_scale[l] + bn_b[l]).reshape(1, H)
        h = _post(a, s1, s2, smn, smx, cnt.reshape(NPAD, 1), h,
                  wsc, wfix, lin2, bias2)

    out = _mlp(h, mlp_W1.T, mlp_b1.reshape(1, H), mlp_W2.T,
               mlp_b2.reshape(1, H))
    return out[:N]


# speedup vs baseline: 71.0283x; 1.4304x over previous
import functools
import numpy as np
import jax
import jax.numpy as jnp
from jax import lax
from jax.experimental import pallas as pl
from jax.experimental.pallas import tpu as pltpu
from jax.experimental.pallas import tpu_sc as plsc

N = 10000
E = 160000
H = 128
L = 4
T = 4
F = 128
TF = T * F
FO = 32
NW = 32
NPW = 320
NPAD = NW * NPW
NCH = 4
CW = TF // NCH
NHALF = 2
NPH = NPW // NHALF
KB = 4000
EB = 128
CAP = 16384
EPAD = ((E + CAP - 1) // CAP) * CAP
AVG_LOG = float(np.log(17.0))
BN_EPS = 1e-5
FINF = 3.0e38


def _wid():
    return lax.axis_index("s") * 2 + lax.axis_index("c")


def _sc_mesh():
    return plsc.VectorSubcoreMesh(core_axis_name="c", subcore_axis_name="s")


def _csr_body(keys_hbm, vals_hbm, col_hbm, rp_hbm, cnt_hbm,
              keys_v, vals_v, hist_v, rp_v, cur_v, buf_v, cntf_v):
    wid = _wid()
    lo = wid * NPW
    ones = jnp.ones((16,), jnp.int32)

    def inith(i, _):
        hist_v[pl.ds(i * 16, 16)] = jnp.zeros((16,), jnp.int32)
        return 0
    lax.fori_loop(0, NPW // 16, inith, 0)

    def p1_block(b, _):
        pltpu.sync_copy(keys_hbm.at[pl.ds(b * KB, KB)], keys_v)

        def p1_vec(i, _):
            k = keys_v[pl.ds(i * 16, 16)]
            m = (k >= lo) & (k < lo + NPW)
            kl = jnp.clip(k - lo, 0, NPW - 1)
            plsc.addupdate_scatter(hist_v, [kl], ones, mask=m)
            return 0
        lax.fori_loop(0, KB // 16, p1_vec, 0)
        return 0
    lax.fori_loop(0, E // KB, p1_block, 0)

    def psum(j, carry):
        v = hist_v[pl.ds(j * 16, 16)]
        c = plsc.cumsum(v)
        rp_v[pl.ds(j * 16, 16)] = carry + c - v
        return carry + lax.reduce_max(c, (0,))
    total = lax.fori_loop(0, NPW // 16, psum, jnp.int32(0))
    lane = lax.iota(jnp.int32, 16)
    rp_v[pl.ds(NPW, 16)] = jnp.where(lane == 0, total, 0)

    pltpu.sync_copy(rp_v, rp_hbm.at[wid])

    def cdeg(j, _):
        cntf_v[pl.ds(j * 16, 16)] = hist_v[pl.ds(j * 16, 16)].astype(jnp.float32)
        return 0
    lax.fori_loop(0, NPW // 16, cdeg, 0)
    pltpu.sync_copy(cntf_v, cnt_hbm.at[pl.ds(lo, NPW)])

    cal, _ = plsc.scan_count(jnp.zeros((16,), jnp.int32))
    r0 = lax.reduce_min(cal, (0,))

    nwin = (total + CAP - 1) // CAP

    def window(w, _):
        wbase = w * CAP

        def zero(i, _):
            buf_v[pl.ds(i * 16, 16)] = jnp.zeros((16,), jnp.int32)
            return 0
        lax.fori_loop(0, CAP // 16, zero, 0)

        def rcur(j, _):
            cur_v[pl.ds(j * 16, 16)] = rp_v[pl.ds(j * 16, 16)]
            return 0
        lax.fori_loop(0, 336 // 16, rcur, 0)

        def p2_block(b, _):
            pltpu.sync_copy(keys_hbm.at[pl.ds(b * KB, KB)], keys_v)
            pltpu.sync_copy(vals_hbm.at[pl.ds(b * KB, KB)], vals_v)

            def p2_vec(i, _):
                k = keys_v[pl.ds(i * 16, 16)]
                v = vals_v[pl.ds(i * 16, 16)]
                m = (k >= lo) & (k < lo + NPW)
                kl = jnp.where(m, jnp.clip(k - lo, 0, NPW - 1), NPW)
                rank, lastm = plsc.scan_count(kl, mask=m)
                base = plsc.load_gather(cur_v, [kl], mask=m)
                pos = base + rank - r0
                mw = m & (pos >= wbase) & (pos < wbase + CAP)
                plsc.store_scatter(buf_v, [jnp.clip(pos - wbase, 0, CAP - 1)],
                                   v, mask=mw)
                plsc.store_scatter(cur_v, [kl], pos + 1, mask=lastm & m)
                return 0
            lax.fori_loop(0, KB // 16, p2_vec, 0)
            return 0
        lax.fori_loop(0, E // KB, p2_block, 0)

        pltpu.sync_copy(buf_v, col_hbm.at[wid].at[pl.ds(wbase, CAP)])
        return 0
    lax.fori_loop(0, nwin, window, 0)


def _build_csr(keys, vals):
    fn = pl.kernel(
        _csr_body,
        out_type=[
            jax.ShapeDtypeStruct((NW, EPAD), jnp.int32),
            jax.ShapeDtypeStruct((NW, 336), jnp.int32),
            jax.ShapeDtypeStruct((NPAD,), jnp.float32),
        ],
        mesh=_sc_mesh(),
        compiler_params=pltpu.CompilerParams(needs_layout_passes=False),
        scratch_types=[
            pltpu.VMEM((KB,), jnp.int32),
            pltpu.VMEM((KB,), jnp.int32),
            pltpu.VMEM((NPW,), jnp.int32),
            pltpu.VMEM((336,), jnp.int32),
            pltpu.VMEM((336,), jnp.int32),
            pltpu.VMEM((CAP,), jnp.int32),
            pltpu.VMEM((NPW,), jnp.float32),
        ],
    )
    return fn(keys, vals)


def _seg_body(b3_hbm, col_hbm, rp_hbm, s1_hbm, s2_hbm, mn_hbm, mx_hbm,
              rp_s, colv2, rows2, accS, accQ, accMn, accMx, sem0, sem1):
    wid = _wid()
    pltpu.sync_copy(rp_hbm.at[wid], rp_s)

    def rd(i):
        return rp_s[pl.ds(i, 16)][0]
    NV = CW // 16

    def chunk(ch, _):
        c = ch // NHALF
        half = ch % NHALF
        n_lo = half * NPH
        n_hi = n_lo + NPH
        lo_e = rd(n_lo)
        hi_e = rd(n_hi)

        def issue(b, par):
            sem = sem0 if par == 0 else sem1
            pltpu.sync_copy(col_hbm.at[wid].at[pl.ds(b * EB, EB)],
                            colv2.at[par])
            pltpu.async_copy(b3_hbm.at[c].at[colv2.at[par]],
                             rows2.at[par], sem)

        def wait(par):
            sem = sem0 if par == 0 else sem1
            pltpu.make_async_copy(b3_hbm.at[c].at[colv2.at[par]],
                                  rows2.at[par], sem).wait()

        def initacc(i, _):
            z = jnp.zeros((16,), jnp.float32)
            r = i // NV
            k = (i % NV) * 16
            accS[r, pl.ds(k, 16)] = z
            accQ[r, pl.ds(k, 16)] = z
            accMn[r, pl.ds(k, 16)] = jnp.full((16,), FINF, jnp.float32)
            accMx[r, pl.ds(k, 16)] = jnp.full((16,), -FINF, jnp.float32)
            return 0
        lax.fori_loop(0, NPH * NV, initacc, 0)

        b0 = lo_e // EB
        b1 = (hi_e + EB - 1) // EB

        @pl.when(b1 > b0)
        def _():
            issue(b0, 0)

        def block(eb, n0):
            par = lax.rem(eb - b0, 2)
            e0 = eb * EB
            e1 = jnp.minimum(e0 + EB, hi_e)

            @pl.when(eb + 1 < b1)
            def _():
                @pl.when(par == 0)
                def _():
                    issue(eb + 1, 1)

                @pl.when(par == 1)
                def _():
                    issue(eb + 1, 0)

            @pl.when(par == 0)
            def _():
                wait(0)

            @pl.when(par == 1)
            def _():
                wait(1)

            def node_cond(carry):
                n, done = carry
                return jnp.logical_not(done) & (n < n_hi)

            def node_body(carry):
                n, _ = carry
                na = n - n_lo
                rs = jnp.maximum(rd(n), e0)
                re = jnp.minimum(rd(n + 1), e1)
                a = [accS[na, pl.ds(k * 16, 16)] for k in range(NV)]
                q = [accQ[na, pl.ds(k * 16, 16)] for k in range(NV)]
                mn = [accMn[na, pl.ds(k * 16, 16)] for k in range(NV)]
                mx = [accMx[na, pl.ds(k * 16, 16)] for k in range(NV)]

                def edge(e, st):
                    sa, sq, smn, smx = st
                    r = e - e0
                    v = [rows2[par, r, pl.ds(k * 16, 16)] for k in range(NV)]
                    sa = [sa[k] + v[k] for k in range(NV)]
                    sq = [sq[k] + v[k] * v[k] for k in range(NV)]
                    smn = [jnp.minimum(smn[k], v[k]) for k in range(NV)]
                    smx = [jnp.maximum(smx[k], v[k]) for k in range(NV)]
                    return (sa, sq, smn, smx)
                a, q, mn, mx = lax.fori_loop(rs, jnp.maximum(rs, re), edge,
                                             (a, q, mn, mx))
                for k in range(NV):
                    accS[na, pl.ds(k * 16, 16)] = a[k]
                    accQ[na, pl.ds(k * 16, 16)] = q[k]
                    accMn[na, pl.ds(k * 16, 16)] = mn[k]
                    accMx[na, pl.ds(k * 16, 16)] = mx[k]
                adv = rd(n + 1) <= e1
                return (jnp.where(adv, n + 1, n), jnp.logical_not(adv))

            nfin, _ = lax.while_loop(node_cond, node_body,
                                     (n0, hi_e <= e0))
            return nfin
        lax.fori_loop(b0, b1, block, n_lo)

        lo = wid * NPW + n_lo
        pltpu.sync_copy(accS, s1_hbm.at[c].at[pl.ds(lo, NPH)])
        pltpu.sync_copy(accQ, s2_hbm.at[c].at[pl.ds(lo, NPH)])
        pltpu.sync_copy(accMn, mn_hbm.at[c].at[pl.ds(lo, NPH)])
        pltpu.sync_copy(accMx, mx_hbm.at[c].at[pl.ds(lo, NPH)])
        return 0
    lax.fori_loop(0, NCH * NHALF, chunk, 0)


def _segment_reduce(b3, col, rp):
    fn = pl.kernel(
        _seg_body,
        out_type=[jax.ShapeDtypeStruct((NCH, NPAD, CW), jnp.float32)
                  for _ in range(4)],
        mesh=_sc_mesh(),
        compiler_params=pltpu.CompilerParams(needs_layout_passes=False),
        scratch_types=[
            pltpu.VMEM((336,), jnp.int32),
            pltpu.VMEM((2, EB), jnp.int32),
            pltpu.VMEM((2, EB, CW), jnp.float32),
            pltpu.VMEM((NPH, CW), jnp.float32),
            pltpu.VMEM((NPH, CW), jnp.float32),
            pltpu.VMEM((NPH, CW), jnp.float32),
            pltpu.VMEM((NPH, CW), jnp.float32),
            pltpu.SemaphoreType.DMA,
            pltpu.SemaphoreType.DMA,
        ],
    )
    return fn(b3, col, rp)


RB = 256


def _in_body(x_ref, w_ref, b_ref, o_ref):
    o_ref[...] = jax.nn.relu(
        jnp.dot(x_ref[...], w_ref[...], preferred_element_type=jnp.float32)
        + b_ref[...])


def _input_proj(x, w_t, b):
    return pl.pallas_call(
        _in_body,
        grid=(NPAD // RB,),
        in_specs=[
            pl.BlockSpec((RB, H), lambda i: (i, 0)),
            pl.BlockSpec((H, H), lambda i: (0, 0)),
            pl.BlockSpec((1, H), lambda i: (0, 0)),
        ],
        out_specs=pl.BlockSpec((RB, H), lambda i: (i, 0)),
        out_shape=jax.ShapeDtypeStruct((NPAD, H), jnp.float32),
    )(x, w_t, b)


def _pre_body(h_ref, wa_ref, wb_ref, pb_ref, a_ref, b3_ref):
    h = h_ref[...]
    a_ref[...] = jnp.dot(h, wa_ref[...],
                         preferred_element_type=jnp.float32) + pb_ref[...]
    b3_ref[0] = jnp.dot(h, wb_ref[...], preferred_element_type=jnp.float32)


def _pre_proj(h, wa, wb, pb):
    return pl.pallas_call(
        _pre_body,
        grid=(NPAD // RB, NCH),
        in_specs=[
            pl.BlockSpec((RB, H), lambda i, j: (i, 0)),
            pl.BlockSpec((H, CW), lambda i, j: (0, j)),
            pl.BlockSpec((H, CW), lambda i, j: (0, j)),
            pl.BlockSpec((1, CW), lambda i, j: (0, j)),
        ],
        out_specs=[
            pl.BlockSpec((RB, CW), lambda i, j: (i, j)),
            pl.BlockSpec((1, RB, CW), lambda i, j: (j, i, 0)),
        ],
        out_shape=[
            jax.ShapeDtypeStruct((NPAD, TF), jnp.float32),
            jax.ShapeDtypeStruct((NCH, NPAD, CW), jnp.float32),
        ],
    )(h, wa, wb, pb)


PB = 320


def _post_body(a_ref, s1_ref, s2_ref, mn_ref, mx_ref, cnt_ref, h_ref,
               wsc_ref, wfix_ref, lin_ref, bias_ref, o_ref):
    cnt = cnt_ref[...]
    deg = jnp.maximum(cnt, 1.0)
    he = cnt > 0.0
    logd = jnp.log(deg + 1.0)
    c1 = logd * (1.0 / AVG_LOG)
    c2 = AVG_LOG / logd
    h = h_ref[...]
    outs = []
    CPT = F // CW
    for t in range(T):
        mean_c, mn_c, mx_c, std_c = [], [], [], []
        for cc in range(CPT):
            c = t * CPT + cc
            A = a_ref[:, pl.ds((t * CPT + cc) * CW, CW)]
            S1 = s1_ref[c]
            mean = (cnt * A + S1) / deg
            msq = (cnt * A * A + 2.0 * A * S1 + s2_ref[c]) / deg
            std = jnp.sqrt(jnp.maximum(msq - mean * mean, 0.0) + 1e-5)
            mean_c.append(mean)
            std_c.append(std)
            mn_c.append(jnp.where(he, A + mn_ref[c], 0.0))
            mx_c.append(jnp.where(he, A + mx_ref[c], 0.0))
        agg = jnp.concatenate(mean_c + mn_c + mx_c + std_c, axis=1)
        psc = jnp.dot(agg, wsc_ref[t], preferred_element_type=jnp.float32)
        pfix = jnp.dot(jnp.concatenate([h, agg], axis=1), wfix_ref[t],
                       preferred_element_type=jnp.float32)
        outs.append(pfix + c1 * psc[:, :FO] + c2 * psc[:, FO:])
    out = jnp.concatenate(outs, axis=1)
    o_ref[...] = jax.nn.relu(
        jnp.dot(out, lin_ref[...], preferred_element_type=jnp.float32)
        + bias_ref[...])


def _post(a, s1, s2, mn, mx, cnt2, h, wsc, wfix, lin2, bias2):
    return pl.pallas_call(
        _post_body,
        grid=(NPAD // PB,),
        in_specs=[
            pl.BlockSpec((PB, TF), lambda i: (i, 0)),
            pl.BlockSpec((NCH, PB, CW), lambda i: (0, i, 0)),
            pl.BlockSpec((NCH, PB, CW), lambda i: (0, i, 0)),
            pl.BlockSpec((NCH, PB, CW), lambda i: (0, i, 0)),
            pl.BlockSpec((NCH, PB, CW), lambda i: (0, i, 0)),
            pl.BlockSpec((PB, 1), lambda i: (i, 0)),
            pl.BlockSpec((PB, H), lambda i: (i, 0)),
            pl.BlockSpec((T, 4 * F, 2 * FO), lambda i: (0, 0, 0)),
            pl.BlockSpec((T, H + 4 * F, FO), lambda i: (0, 0, 0)),
            pl.BlockSpec((H, H), lambda i: (0, 0)),
            pl.BlockSpec((1, H), lambda i: (0, 0)),
        ],
        out_specs=pl.BlockSpec((PB, H), lambda i: (i, 0)),
        out_shape=jax.ShapeDtypeStruct((NPAD, H), jnp.float32),
    )(a, s1, s2, mn, mx, cnt2, h, wsc, wfix, lin2, bias2)


def _mlp_body(h_ref, w1_ref, b1_ref, w2_ref, b2_ref, o_ref):
    t = jax.nn.relu(
        jnp.dot(h_ref[...], w1_ref[...], preferred_element_type=jnp.float32)
        + b1_ref[...])
    o_ref[...] = jnp.dot(t, w2_ref[...],
                         preferred_element_type=jnp.float32) + b2_ref[...]


def _mlp(h, w1t, b1, w2t, b2):
    return pl.pallas_call(
        _mlp_body,
        grid=(NPAD // RB,),
        in_specs=[
            pl.BlockSpec((RB, H), lambda i: (i, 0)),
            pl.BlockSpec((H, H), lambda i: (0, 0)),
            pl.BlockSpec((1, H), lambda i: (0, 0)),
            pl.BlockSpec((H, H), lambda i: (0, 0)),
            pl.BlockSpec((1, H), lambda i: (0, 0)),
        ],
        out_specs=pl.BlockSpec((RB, H), lambda i: (i, 0)),
        out_shape=jax.ShapeDtypeStruct((NPAD, H), jnp.float32),
    )(h, w1t, b1, w2t, b2)


def kernel(x, edge_index, W_in, b_in, pre_W, pre_b, post_W, post_b,
           lin_W, lin_b, bn_w, bn_b, mlp_W1, mlp_b1, mlp_W2, mlp_b2):
    x = x.astype(jnp.float32)
    xp = jnp.pad(x, ((0, NPAD - N), (0, 0)))
    src = edge_index[0].astype(jnp.int32)
    dst = edge_index[1].astype(jnp.int32)

    col_f, rp_f, cnt_f = _build_csr(dst, src)
    col_b, rp_b, cnt_b = _build_csr(src, dst)

    h = _input_proj(xp, W_in.T, b_in.reshape(1, H))

    dirs = ['f', 'f', 'b', 'b']
    bn_scale = (bn_w / np.sqrt(1.0 + BN_EPS)).astype(jnp.float32)
    for l in range(L):
        col, rp, cnt = (col_f, rp_f, cnt_f) if dirs[l] == 'f' else \
                       (col_b, rp_b, cnt_b)
        preW = pre_W[l]
        wa = preW[:, :, :F].transpose(2, 0, 1).reshape(H, TF)
        wb = preW[:, :, F:].transpose(2, 0, 1).reshape(H, TF)
        pb = pre_b[l].reshape(1, TF)
        a, b3 = _pre_proj(h, wa, wb, pb)
        s1, s2, smn, smx = _segment_reduce(b3, col, rp)

        pw = post_W[l]
        wx = pw[:, :, :F]
        wamp = pw[:, :, F:5 * F]
        watt = pw[:, :, 5 * F:9 * F]
        wid_ = pw[:, :, 9 * F:]
        wsc = jnp.concatenate([wamp.transpose(0, 2, 1),
                               watt.transpose(0, 2, 1)], axis=2)
        wfix = jnp.concatenate([wx, wid_], axis=2).transpose(0, 2, 1)
        bias = (post_b[l].reshape(-1) @ lin_W[l].T + lin_b[l])
        lin2 = lin_W[l].T * bn_scale[l][None, :]
        bias2 = (bias * bn_scale[l] + bn_b[l]).reshape(1, H)
        h = _post(a, s1, s2, smn, smx, cnt.reshape(NPAD, 1), h,
                  wsc, wfix, lin2, bias2)

    out = _mlp(h, mlp_W1.T, mlp_b1.reshape(1, H), mlp_W2.T,
               mlp_b2.reshape(1, H))
    return out[:N]
